# baseline (speedup 1.0000x reference)
import functools

import jax
import jax.numpy as jnp
from jax import lax
from jax.experimental import pallas as pl
from jax.experimental.pallas import tpu as pltpu
from jax.experimental.pallas import tpu_sc as plsc

K = 8192
D = 256
B = 4096

BB = 2048
BK = 1024
NB = B // BB
NK = K // BK

_NC = 2
_NS = 16
_NW = _NC * _NS
_BPW = B // _NW


_NL = 128


def _mm_argmax(z_ref, ea_ref, ev_ref, out_ref, idx_ref, ebuf_ref, max_ref,
               arg_ref):
    c = pl.program_id(0)
    bk = pl.program_id(1)
    bb = pl.program_id(2)

    @pl.when(c == 0)
    def _():
        ebuf_ref[...] = ea_ref[...]

    @pl.when(c == 1)
    def _():
        ebuf_ref[...] = ev_ref[...]

    blk = lax.dot_general(z_ref[...], ebuf_ref[...], (((1,), (1,)), ((), ())),
                          preferred_element_type=jnp.float32)
    out_ref[...] = blk

    nj = BK // _NL
    M = blk[:, :_NL]
    for j in range(1, nj):
        M = jnp.maximum(M, blk[:, j * _NL:(j + 1) * _NL])
    J = jnp.full((BB, _NL), nj, jnp.int32)
    for j in range(nj - 1, -1, -1):
        J = jnp.where(blk[:, j * _NL:(j + 1) * _NL] == M, j, J)
    lane = lax.broadcasted_iota(jnp.int32, (BB, _NL), 1)
    gcol = bk * BK + J * _NL + lane
    rows = pl.ds(bb * BB, BB)

    @pl.when(bk == 0)
    def _():
        max_ref[rows, :] = M
        arg_ref[rows, :] = gcol

    @pl.when(bk > 0)
    def _():
        cur = max_ref[rows, :]
        gt = M > cur
        max_ref[rows, :] = jnp.maximum(cur, M)
        arg_ref[rows, :] = jnp.where(gt, gcol, arg_ref[rows, :])

    @pl.when(bk == NK - 1)
    def _():
        Mr = max_ref[rows, :]
        Cr = arg_ref[rows, :]
        gmax = jnp.max(Mr, axis=1, keepdims=True)
        idx_ref[0] = jnp.min(jnp.where(Mr == gmax, Cr, K),
                             axis=1, keepdims=True)


def _logits_and_idx(z_e_x, emb_a, emb_v):
    return pl.pallas_call(
        _mm_argmax,
        grid=(2, NK, NB),
        in_specs=[
            pl.BlockSpec((BB, D), lambda c, k, b: (b, c)),
            pl.BlockSpec((BK, D), lambda c, k, b: (jnp.where(c == 0, k, 0), 0)),
            pl.BlockSpec((BK, D), lambda c, k, b: (jnp.where(c == 1, k, 0), 0)),
        ],
        out_specs=[
            pl.BlockSpec((BB, BK), lambda c, k, b: (b, c * NK + k)),
            pl.BlockSpec((1, BB, 1), lambda c, k, b: (c, b, 0)),
        ],
        out_shape=[
            jax.ShapeDtypeStruct((B, 2 * K), jnp.float32),
            jax.ShapeDtypeStruct((2, B, 1), jnp.int32),
        ],
        scratch_shapes=[
            pltpu.VMEM((BK, D), jnp.float32),
            pltpu.VMEM((B, _NL), jnp.float32),
            pltpu.VMEM((B, _NL), jnp.int32),
        ],
        compiler_params=pltpu.CompilerParams(
            dimension_semantics=("arbitrary", "arbitrary", "arbitrary"),
            vmem_limit_bytes=63 * 1024 * 1024),
    )(z_e_x, emb_a, emb_v)


def _sc_gather(emb_a, emb_v, idx_flat):
    mesh = plsc.VectorSubcoreMesh(core_axis_name="c", subcore_axis_name="s")

    @functools.partial(
        pl.kernel, mesh=mesh,
        out_type=jax.ShapeDtypeStruct((2 * B, D), jnp.float32),
        scratch_types=[
            pltpu.VMEM((_BPW,), jnp.int32),
            pltpu.VMEM((_BPW, D), jnp.float32),
            pltpu.SemaphoreType.DMA,
        ],
    )
    def gather(ea_hbm, ev_hbm, idx_hbm, out_hbm, idx_v, rows_v, sem):
        wid = lax.axis_index("s") * _NC + lax.axis_index("c")
        for t, tab in ((0, ea_hbm), (1, ev_hbm)):
            base = t * B + wid * _BPW
            pltpu.sync_copy(idx_hbm.at[pl.ds(base, _BPW)], idx_v)
            pltpu.async_copy(tab.at[idx_v], rows_v, sem).wait()
            pltpu.sync_copy(rows_v, out_hbm.at[pl.ds(base, _BPW)])

    return gather(emb_a, emb_v, idx_flat)


def kernel(z_e_x, emb_a, emb_v):
    logits, idx = _logits_and_idx(z_e_x, emb_a, emb_v)
    idx_flat = idx.reshape(2 * B)
    rows = _sc_gather(emb_a, emb_v, idx_flat)
    z_q = jnp.concatenate([rows[:B], rows[B:]], axis=-1)
    return z_q, z_q, logits


# speedup vs baseline: 1.2660x; 1.2660x over previous
import functools

import jax
import jax.numpy as jnp
from jax import lax
from jax.experimental import pallas as pl
from jax.experimental.pallas import tpu as pltpu
from jax.experimental.pallas import tpu_sc as plsc

K = 8192
D = 256
B = 4096

BB = 4096
BK = 1024
NB = B // BB
NK = K // BK

_NC = 2
_NS = 16
_NW = _NC * _NS
_BPW = B // _NW


_NL = 128


def _mm_argmax(z_ref, ea_ref, ev_ref, out_ref, idx_ref, ebuf_ref, max_ref,
               arg_ref):
    c = pl.program_id(0)
    bk = pl.program_id(1)
    bb = pl.program_id(2)

    @pl.when(c == 0)
    def _():
        ebuf_ref[...] = ea_ref[...]

    @pl.when(c == 1)
    def _():
        ebuf_ref[...] = ev_ref[...]

    blk = lax.dot_general(z_ref[...], ebuf_ref[...], (((1,), (1,)), ((), ())),
                          preferred_element_type=jnp.float32)
    out_ref[...] = blk

    nj = BK // _NL
    M = blk[:, :_NL]
    for j in range(1, nj):
        M = jnp.maximum(M, blk[:, j * _NL:(j + 1) * _NL])
    J = jnp.full((BB, _NL), nj, jnp.int32)
    for j in range(nj - 1, -1, -1):
        J = jnp.where(blk[:, j * _NL:(j + 1) * _NL] == M, j, J)
    lane = lax.broadcasted_iota(jnp.int32, (BB, _NL), 1)
    gcol = bk * BK + J * _NL + lane
    rows = pl.ds(bb * BB, BB)

    @pl.when(bk == 0)
    def _():
        max_ref[rows, :] = M
        arg_ref[rows, :] = gcol

    @pl.when(bk > 0)
    def _():
        cur = max_ref[rows, :]
        gt = M > cur
        max_ref[rows, :] = jnp.maximum(cur, M)
        arg_ref[rows, :] = jnp.where(gt, gcol, arg_ref[rows, :])

    @pl.when(bk == NK - 1)
    def _():
        Mr = max_ref[rows, :]
        Cr = arg_ref[rows, :]
        gmax = jnp.max(Mr, axis=1, keepdims=True)
        idx_ref[0] = jnp.min(jnp.where(Mr == gmax, Cr, K),
                             axis=1, keepdims=True)


def _logits_and_idx(z_e_x, emb_a, emb_v):
    return pl.pallas_call(
        _mm_argmax,
        grid=(2, NK, NB),
        in_specs=[
            pl.BlockSpec((BB, D), lambda c, k, b: (b, c)),
            pl.BlockSpec((BK, D), lambda c, k, b: (jnp.where(c == 0, k, 0), 0)),
            pl.BlockSpec((BK, D), lambda c, k, b: (jnp.where(c == 1, k, 0), 0)),
        ],
        out_specs=[
            pl.BlockSpec((BB, BK), lambda c, k, b: (b, c * NK + k)),
            pl.BlockSpec((1, BB, 1), lambda c, k, b: (c, b, 0)),
        ],
        out_shape=[
            jax.ShapeDtypeStruct((B, 2 * K), jnp.float32),
            jax.ShapeDtypeStruct((2, B, 1), jnp.int32),
        ],
        scratch_shapes=[
            pltpu.VMEM((BK, D), jnp.float32),
            pltpu.VMEM((B, _NL), jnp.float32),
            pltpu.VMEM((B, _NL), jnp.int32),
        ],
        compiler_params=pltpu.CompilerParams(
            dimension_semantics=("arbitrary", "arbitrary", "arbitrary"),
            vmem_limit_bytes=63 * 1024 * 1024),
    )(z_e_x, emb_a, emb_v)


def _sc_gather(emb_a, emb_v, idx_flat):
    mesh = plsc.VectorSubcoreMesh(core_axis_name="c", subcore_axis_name="s")

    @functools.partial(
        pl.kernel, mesh=mesh,
        out_type=jax.ShapeDtypeStruct((B, 2 * D), jnp.float32),
        scratch_types=[
            pltpu.VMEM((_BPW,), jnp.int32),
            pltpu.VMEM((_BPW, D), jnp.float32),
            pltpu.SemaphoreType.DMA,
        ],
    )
    def gather(ea_hbm, ev_hbm, idx_hbm, out_hbm, idx_v, rows_v, sem):
        wid = lax.axis_index("s") * _NC + lax.axis_index("c")
        for t, tab in ((0, ea_hbm), (1, ev_hbm)):
            base = t * B + wid * _BPW
            pltpu.sync_copy(idx_hbm.at[pl.ds(base, _BPW)], idx_v)
            pltpu.async_copy(tab.at[idx_v], rows_v, sem).wait()
            pltpu.sync_copy(
                rows_v,
                out_hbm.at[pl.ds(wid * _BPW, _BPW), pl.ds(t * D, D)])

    return gather(emb_a, emb_v, idx_flat)


def kernel(z_e_x, emb_a, emb_v):
    logits, idx = _logits_and_idx(z_e_x, emb_a, emb_v)
    idx_flat = idx.reshape(2 * B)
    z_q = _sc_gather(emb_a, emb_v, idx_flat)
    return z_q, z_q, logits


# speedup vs baseline: 1.3155x; 1.0390x over previous
import functools

import jax
import jax.numpy as jnp
from jax import lax
from jax.experimental import pallas as pl
from jax.experimental.pallas import tpu as pltpu
from jax.experimental.pallas import tpu_sc as plsc

K = 8192
D = 256
B = 4096

BB = 4096
BK = 1024
NB = B // BB
NK = K // BK

_NC = 2
_NS = 16
_NW = _NC * _NS
_BPW = B // _NW


_NL = 128


def _mm_argmax(z_ref, ea_ref, ev_ref, out_ref, idx_ref, ebuf_ref, max_ref,
               arg_ref):
    c = pl.program_id(0)
    bk = pl.program_id(1)
    bb = pl.program_id(2)

    @pl.when(c == 0)
    def _():
        ebuf_ref[...] = ea_ref[...]

    @pl.when(c == 1)
    def _():
        ebuf_ref[...] = ev_ref[...]

    blk = lax.dot_general(z_ref[...], ebuf_ref[...], (((1,), (1,)), ((), ())),
                          preferred_element_type=jnp.float32)
    out_ref[...] = blk

    nj = BK // _NL
    M = blk[:, :_NL]
    for j in range(1, nj):
        M = jnp.maximum(M, blk[:, j * _NL:(j + 1) * _NL])
    J = jnp.full((BB, _NL), nj, jnp.int32)
    for j in range(nj - 1, -1, -1):
        J = jnp.where(blk[:, j * _NL:(j + 1) * _NL] == M, j, J)
    lane = lax.broadcasted_iota(jnp.int32, (BB, _NL), 1)
    gcol = bk * BK + J * _NL + lane
    rows = pl.ds(bb * BB, BB)

    @pl.when(bk == 0)
    def _():
        max_ref[rows, :] = M
        arg_ref[rows, :] = gcol

    @pl.when(bk > 0)
    def _():
        cur = max_ref[rows, :]
        gt = M > cur
        max_ref[rows, :] = jnp.maximum(cur, M)
        arg_ref[rows, :] = jnp.where(gt, gcol, arg_ref[rows, :])

    @pl.when(bk == NK - 1)
    def _():
        Mr = max_ref[rows, :]
        Cr = arg_ref[rows, :]
        gmax = jnp.max(Mr, axis=1, keepdims=True)
        idx_ref[0] = jnp.min(jnp.where(Mr == gmax, Cr, K),
                             axis=1, keepdims=True)


def _logits_and_idx(z_e_x, emb_a, emb_v):
    return pl.pallas_call(
        _mm_argmax,
        grid=(2, NK, NB),
        in_specs=[
            pl.BlockSpec((BB, D), lambda c, k, b: (b, c)),
            pl.BlockSpec((BK, D), lambda c, k, b: (jnp.where(c == 0, k, 0), 0)),
            pl.BlockSpec((BK, D), lambda c, k, b: (jnp.where(c == 1, k, 0), 0)),
        ],
        out_specs=[
            pl.BlockSpec((BB, BK), lambda c, k, b: (b, c * NK + k)),
            pl.BlockSpec((1, BB, 1), lambda c, k, b: (c, b, 0)),
        ],
        out_shape=[
            jax.ShapeDtypeStruct((B, 2 * K), jnp.float32),
            jax.ShapeDtypeStruct((2, B, 1), jnp.int32),
        ],
        scratch_shapes=[
            pltpu.VMEM((BK, D), jnp.float32),
            pltpu.VMEM((B, _NL), jnp.float32),
            pltpu.VMEM((B, _NL), jnp.int32),
        ],
        compiler_params=pltpu.CompilerParams(
            dimension_semantics=("arbitrary", "arbitrary", "arbitrary"),
            vmem_limit_bytes=63 * 1024 * 1024),
    )(z_e_x, emb_a, emb_v)


def _sc_gather(emb_a, emb_v, idx_flat):
    mesh = plsc.VectorSubcoreMesh(core_axis_name="c", subcore_axis_name="s")

    @functools.partial(
        pl.kernel, mesh=mesh,
        out_type=[jax.ShapeDtypeStruct((B, 2 * D), jnp.float32),
                  jax.ShapeDtypeStruct((B, 2 * D), jnp.float32)],
        scratch_types=[
            pltpu.VMEM((_BPW,), jnp.int32),
            pltpu.VMEM((_BPW, D), jnp.float32),
            pltpu.SemaphoreType.DMA,
        ],
    )
    def gather(ea_hbm, ev_hbm, idx_hbm, st_hbm, q_hbm, idx_v, rows_v, sem):
        wid = lax.axis_index("s") * _NC + lax.axis_index("c")
        for t, tab in ((0, ea_hbm), (1, ev_hbm)):
            base = t * B + wid * _BPW
            pltpu.sync_copy(idx_hbm.at[pl.ds(base, _BPW)], idx_v)
            pltpu.async_copy(tab.at[idx_v], rows_v, sem).wait()
            dst = (pl.ds(wid * _BPW, _BPW), pl.ds(t * D, D))
            pltpu.sync_copy(rows_v, st_hbm.at[dst[0], dst[1]])
            pltpu.sync_copy(rows_v, q_hbm.at[dst[0], dst[1]])

    return gather(emb_a, emb_v, idx_flat)


def kernel(z_e_x, emb_a, emb_v):
    logits, idx = _logits_and_idx(z_e_x, emb_a, emb_v)
    idx_flat = idx.reshape(2 * B)
    z_q_st, z_q = _sc_gather(emb_a, emb_v, idx_flat)
    return z_q_st, z_q, logits


# speedup vs baseline: 1.3374x; 1.0167x over previous
import functools

import jax
import jax.numpy as jnp
from jax import lax
from jax.experimental import pallas as pl
from jax.experimental.pallas import tpu as pltpu
from jax.experimental.pallas import tpu_sc as plsc

K = 8192
D = 256
B = 4096

BB = 4096
BK = 1024
NB = B // BB
NK = K // BK

_NC = 2
_NS = 16
_NW = _NC * _NS
_BPW = B // _NW


_NL = 128


def _mm_argmax(z_ref, ea_ref, ev_ref, out_ref, idx_ref, ebuf_ref, max_ref,
               arg_ref):
    c = pl.program_id(0)
    bk = pl.program_id(1)
    bb = pl.program_id(2)

    @pl.when(c == 0)
    def _():
        ebuf_ref[...] = ea_ref[...]

    @pl.when(c == 1)
    def _():
        ebuf_ref[...] = ev_ref[...]

    blk = lax.dot_general(z_ref[...], ebuf_ref[...], (((1,), (1,)), ((), ())),
                          preferred_element_type=jnp.float32)
    out_ref[...] = blk

    nj = BK // _NL
    M = blk[:, :_NL]
    for j in range(1, nj):
        M = jnp.maximum(M, blk[:, j * _NL:(j + 1) * _NL])
    J = jnp.full((BB, _NL), nj, jnp.int32)
    for j in range(nj - 1, -1, -1):
        J = jnp.where(blk[:, j * _NL:(j + 1) * _NL] == M, j, J)
    lane = lax.broadcasted_iota(jnp.int32, (BB, _NL), 1)
    gcol = bk * BK + J * _NL + lane
    rows = pl.ds(bb * BB, BB)

    @pl.when(bk == 0)
    def _():
        max_ref[rows, :] = M
        arg_ref[rows, :] = gcol

    @pl.when(bk > 0)
    def _():
        cur = max_ref[rows, :]
        gt = M > cur
        max_ref[rows, :] = jnp.maximum(cur, M)
        arg_ref[rows, :] = jnp.where(gt, gcol, arg_ref[rows, :])

    @pl.when(bk == NK - 1)
    def _():
        Mr = max_ref[rows, :]
        Cr = arg_ref[rows, :]
        gmax = jnp.max(Mr, axis=1, keepdims=True)
        idx_ref[0] = jnp.min(jnp.where(Mr == gmax, Cr, K),
                             axis=1, keepdims=True)


def _logits_and_idx(z_e_x, emb_a, emb_v):
    return pl.pallas_call(
        _mm_argmax,
        grid=(2, NK, NB),
        in_specs=[
            pl.BlockSpec((BB, D), lambda c, k, b: (b, c)),
            pl.BlockSpec((BK, D), lambda c, k, b: (jnp.where(c == 0, k, 0), 0)),
            pl.BlockSpec((BK, D), lambda c, k, b: (jnp.where(c == 1, k, 0), 0)),
        ],
        out_specs=[
            pl.BlockSpec((BB, BK), lambda c, k, b: (b, c * NK + k)),
            pl.BlockSpec((1, BB, 1), lambda c, k, b: (c, b, 0)),
        ],
        out_shape=[
            jax.ShapeDtypeStruct((B, 2 * K), jnp.float32),
            jax.ShapeDtypeStruct((2, B, 1), jnp.int32),
        ],
        scratch_shapes=[
            pltpu.VMEM((BK, D), jnp.float32),
            pltpu.VMEM((B, _NL), jnp.float32),
            pltpu.VMEM((B, _NL), jnp.int32),
        ],
        compiler_params=pltpu.CompilerParams(
            dimension_semantics=("arbitrary", "arbitrary", "arbitrary"),
            vmem_limit_bytes=63 * 1024 * 1024),
    )(z_e_x, emb_a, emb_v)


def _sc_gather(emb_a, emb_v, idx_flat):
    mesh = plsc.VectorSubcoreMesh(core_axis_name="c", subcore_axis_name="s")

    @functools.partial(
        pl.kernel, mesh=mesh,
        out_type=[jax.ShapeDtypeStruct((B, 2 * D), jnp.float32),
                  jax.ShapeDtypeStruct((B, 2 * D), jnp.float32)],
        scratch_types=[
            pltpu.VMEM((_BPW,), jnp.int32),
            pltpu.VMEM((_BPW,), jnp.int32),
            pltpu.VMEM((_BPW, D), jnp.float32),
            pltpu.VMEM((_BPW, D), jnp.float32),
            pltpu.SemaphoreType.DMA,
            pltpu.SemaphoreType.DMA,
        ],
    )
    def gather(ea_hbm, ev_hbm, idx_hbm, st_hbm, q_hbm, idx_a, idx_b, rows_a,
               rows_b, sem_g, sem_w):
        wid = lax.axis_index("s") * _NC + lax.axis_index("c")
        base = wid * _BPW
        pltpu.sync_copy(idx_hbm.at[pl.ds(base, _BPW)], idx_a)
        pltpu.sync_copy(idx_hbm.at[pl.ds(B + base, _BPW)], idx_b)
        ga = pltpu.async_copy(ea_hbm.at[idx_a], rows_a, sem_g)
        gb = pltpu.async_copy(ev_hbm.at[idx_b], rows_b, sem_g)
        dst_r = pl.ds(base, _BPW)
        ga.wait()
        w1 = pltpu.async_copy(rows_a, st_hbm.at[dst_r, pl.ds(0, D)], sem_w)
        w2 = pltpu.async_copy(rows_a, q_hbm.at[dst_r, pl.ds(0, D)], sem_w)
        gb.wait()
        w3 = pltpu.async_copy(rows_b, st_hbm.at[dst_r, pl.ds(D, D)], sem_w)
        w4 = pltpu.async_copy(rows_b, q_hbm.at[dst_r, pl.ds(D, D)], sem_w)
        w1.wait()
        w2.wait()
        w3.wait()
        w4.wait()

    return gather(emb_a, emb_v, idx_flat)


def kernel(z_e_x, emb_a, emb_v):
    logits, idx = _logits_and_idx(z_e_x, emb_a, emb_v)
    idx_flat = idx.reshape(2 * B)
    z_q_st, z_q = _sc_gather(emb_a, emb_v, idx_flat)
    return z_q_st, z_q, logits
